# Initial kernel scaffold; baseline (speedup 1.0000x reference)
import jax
import jax.numpy as jnp
from jax.experimental import pallas as pl


def kernel(x, edge_index, W1, b1, W2, b2, W3, b3, W4, b4):
    raise NotImplementedError("write your pallas kernel here")


# speedup vs baseline: 3.1167x; 3.1167x over previous
import functools

import jax
import jax.numpy as jnp
from jax import lax
from jax.experimental import pallas as pl
from jax.experimental.pallas import tpu as pltpu
from jax.experimental.pallas import tpu_sc as plsc

_N = 10000
_E = 320000
_D = 128
_H4 = 64
_K = 64

_NC = 2
_NS = 16
_CH = 128
_DNCH = 79
_DN = 10240
_SCH = 128
_SNCH = 158
_EPAD = _NS * _SNCH * _SCH
_ZB = 40
_NH = _N // _NC
_TRASH = _NH
_AGGR = 5120
_ABN = _AGGR // _ZB

_f32 = jnp.float32
_i32 = jnp.int32


def _scatter_body(edges_hbm, hs_hbm, out_hbm, idxv, rows, zb, aggsh, sems):
    c = lax.axis_index("c")
    s = lax.axis_index("s")
    pltpu.sync_copy(edges_hbm.at[s], idxv.at[pl.ds(0, _SNCH)])
    pltpu.sync_copy(edges_hbm.at[_NS + s], idxv.at[pl.ds(160, _SNCH)])
    pltpu.sync_copy(hs_hbm.at[pl.ds(_N, _ZB)], zb)

    base = c * _NH

    def remap(j, _):
        r = 160 + j
        for q in range(_SCH // 16):
            sl = pl.ds(q * 16, 16)
            v = idxv[r, sl] - base
            ok = (v >= 0) & (v < _NH)
            idxv[r, sl] = jnp.where(ok, v, _TRASH)
        return 0

    lax.fori_loop(0, _SNCH, remap, 0)

    for k in range(_ABN // _NS):
        b = s * (_ABN // _NS) + k
        pltpu.sync_copy(zb, aggsh.at[pl.ds(b * _ZB, _ZB)])
    plsc.subcore_barrier()

    pltpu.async_copy(hs_hbm.at[idxv.at[0]], rows.at[0], sems.at[0])
    pltpu.async_copy(hs_hbm.at[idxv.at[1]], rows.at[1], sems.at[1])

    def pair(i, _):
        j0 = 2 * i
        j1 = 2 * i + 1
        pltpu.make_async_copy(
            hs_hbm.at[idxv.at[j0]], rows.at[0], sems.at[0]).wait()
        pltpu.sync_copy(rows.at[0], aggsh.at[idxv.at[160 + j0]], add=True)
        pltpu.async_copy(hs_hbm.at[idxv.at[j0 + 2]], rows.at[0], sems.at[0])
        pltpu.make_async_copy(
            hs_hbm.at[idxv.at[j1]], rows.at[1], sems.at[1]).wait()
        pltpu.sync_copy(rows.at[1], aggsh.at[idxv.at[160 + j1]], add=True)
        pltpu.async_copy(hs_hbm.at[idxv.at[j1 + 2]], rows.at[1], sems.at[1])
        return 0

    lax.fori_loop(0, _SNCH // 2 - 1, pair, 0)
    jl0 = _SNCH - 2
    jl1 = _SNCH - 1
    pltpu.make_async_copy(
        hs_hbm.at[idxv.at[jl0]], rows.at[0], sems.at[0]).wait()
    pltpu.sync_copy(rows.at[0], aggsh.at[idxv.at[160 + jl0]], add=True)
    pltpu.make_async_copy(
        hs_hbm.at[idxv.at[jl1]], rows.at[1], sems.at[1]).wait()
    pltpu.sync_copy(rows.at[1], aggsh.at[idxv.at[160 + jl1]], add=True)
    plsc.subcore_barrier()

    for k in range(_ABN // _NS):
        b = s * (_ABN // _NS) + k
        pltpu.sync_copy(aggsh.at[pl.ds(b * _ZB, _ZB)], zb)
        pltpu.sync_copy(zb, out_hbm.at[c * _ABN + b])


@functools.cache
def _sc_kernels():
    mesh = plsc.VectorSubcoreMesh(core_axis_name="c", subcore_axis_name="s")
    scatter = pl.kernel(
        _scatter_body,
        out_type=jax.ShapeDtypeStruct((_NC * _ABN, _ZB, _D), _f32),
        mesh=mesh,
        scratch_types=[
            pltpu.VMEM((320, _SCH), _i32),
            pltpu.VMEM((2, _SCH, _D), _f32),
            pltpu.VMEM((_ZB, _D), _f32),
            pltpu.VMEM_SHARED((_AGGR, _D), _f32),
            pltpu.SemaphoreType.DMA((2,)),
        ],
    )
    return scatter


_BR = 1000
_GRID = (_N // _BR,)


def _tc_pre_body(dop, dip, x, hs1, nout, nin):
    do = dop[0, :, 0:1] + 1.0
    di = dip[0, :, 0:1] + 1.0
    no = lax.rsqrt(do)
    ni = lax.rsqrt(di)
    nout[...] = no
    nin[...] = ni
    hs1[...] = x[...] * no


_tc_pre = pl.pallas_call(
    _tc_pre_body,
    grid=_GRID,
    in_specs=[
        pl.BlockSpec((1, _BR, _D), lambda i: (i // 5, i % 5, 0)),
        pl.BlockSpec((1, _BR, _D), lambda i: (i // 5, i % 5, 0)),
        pl.BlockSpec((_BR, _D), lambda i: (i, 0)),
    ],
    out_specs=[
        pl.BlockSpec((_BR, _D), lambda i: (i, 0)),
        pl.BlockSpec((_BR, 1), lambda i: (i, 0)),
        pl.BlockSpec((_BR, 1), lambda i: (i, 0)),
    ],
    out_shape=[
        jax.ShapeDtypeStruct((_N, _D), _f32),
        jax.ShapeDtypeStruct((_N, 1), _f32),
        jax.ShapeDtypeStruct((_N, 1), _f32),
    ],
)


def _tc_mid_body(p, hs, nin, nout, W, b, o):
    agg = (p[0] + hs[...]) * nin[...]
    h = jnp.maximum(jnp.dot(agg, W[...], preferred_element_type=_f32)
                    + b[...], 0.0)
    o[...] = h * nout[...]


_tc_mid = pl.pallas_call(
    _tc_mid_body,
    grid=_GRID,
    in_specs=[
        pl.BlockSpec((1, _BR, _D), lambda i: (i // 5, i % 5, 0)),
        pl.BlockSpec((_BR, _D), lambda i: (i, 0)),
        pl.BlockSpec((_BR, 1), lambda i: (i, 0)),
        pl.BlockSpec((_BR, 1), lambda i: (i, 0)),
        pl.BlockSpec((_D, _D), lambda i: (0, 0)),
        pl.BlockSpec((1, _D), lambda i: (0, 0)),
    ],
    out_specs=pl.BlockSpec((_BR, _D), lambda i: (i, 0)),
    out_shape=jax.ShapeDtypeStruct((_N, _D), _f32),
)


def _tc_final_body(p, hs, nin, W4, b4, out, hscr):
    pc = jnp.concatenate([p[0, :_NH], p[1, :_NH]], axis=0)
    agg = (pc + hs[...]) * nin[...]
    h = jnp.maximum(jnp.dot(agg, W4[...], preferred_element_type=_f32)
                    + b4[...], 0.0)
    hscr[...] = h
    m = jnp.max(h, axis=1, keepdims=True)
    iota = lax.broadcasted_iota(_i32, (_N, 1), 0)
    neg = _f32(-jnp.inf)

    mw = m
    rows = []
    for _ in range(_K):
        mx = jnp.max(mw)
        idx = jnp.min(jnp.where(mw == mx, iota, _N))
        rows.append(hscr[pl.ds(idx, 1), :])
        mw = jnp.where(iota == idx, neg, mw)
    v = jnp.concatenate(rows, axis=0)

    lane = lax.broadcasted_iota(_i32, (_K, _H4), 1)
    even = (lane % 2) == 0
    for pss in range(_H4):
        r = jnp.concatenate([v[:, 1:], v[:, -1:]], axis=1)
        l = jnp.concatenate([v[:, 0:1], v[:, :-1]], axis=1)
        if pss % 2 == 0:
            v = jnp.where(even, jnp.minimum(v, r), jnp.maximum(v, l))
        else:
            start = jnp.logical_not(even) & (lane < _H4 - 1)
            end = even & (lane > 0)
            v = jnp.where(start, jnp.minimum(v, r),
                          jnp.where(end, jnp.maximum(v, l), v))
    out[...] = v


_tc_final = pl.pallas_call(
    _tc_final_body,
    out_shape=jax.ShapeDtypeStruct((_K, _H4), _f32),
    scratch_shapes=[pltpu.VMEM((_N, _H4), _f32)],
)


def kernel(x, edge_index, W1, b1, W2, b2, W3, b3, W4, b4):
    npad = _EPAD - _E
    srcp = jnp.concatenate([edge_index[0], jnp.zeros((npad,), _i32)])
    dstp = jnp.concatenate([edge_index[1], jnp.full((npad,), _N, _i32)])
    edges_s = jnp.concatenate([srcp, dstp]).reshape(2 * _NS, _SNCH, _SCH)
    edges_w = jnp.concatenate([dstp, srcp]).reshape(2 * _NS, _SNCH, _SCH)

    _sc_scatter = _sc_kernels()
    zrows = jnp.zeros((_ZB, _D), _f32)

    def _scat(hs, edges):
        return _sc_scatter(
            edges, jnp.concatenate([hs, zrows])).reshape(_NC, _AGGR, _D)

    ones_nd = jnp.ones((_N, _D), _f32)
    dgi = _scat(ones_nd, edges_s)
    dgo = _scat(ones_nd, edges_w)
    hs1, nout, nin = _tc_pre(dgo, dgi, x)

    p1 = _scat(hs1, edges_s)
    hs2 = _tc_mid(p1, hs1, nin, nout, W1, b1.reshape(1, _D))

    p2 = _scat(hs2, edges_s)
    hs3 = _tc_mid(p2, hs2, nin, nout, W2, b2.reshape(1, _D))

    p3 = _scat(hs3, edges_s)
    hs4 = _tc_mid(p3, hs3, nin, nout, W3, b3.reshape(1, _D))

    p4 = _scat(hs4, edges_s)
    pooled = _tc_final(p4, hs4, nin, W4, b4.reshape(1, _H4))
    return pooled.reshape(1, _K * _H4)


# speedup vs baseline: 3.7910x; 1.2164x over previous
import functools

import jax
import jax.numpy as jnp
from jax import lax
from jax.experimental import pallas as pl
from jax.experimental.pallas import tpu as pltpu
from jax.experimental.pallas import tpu_sc as plsc

_N = 10000
_E = 320000
_D = 128
_H4 = 64
_K = 64

_NC = 2
_NS = 16
_CH = 128
_DNCH = 79
_DN = 10240
_SCH = 128
_SNCH = 158
_EPAD = _NS * _SNCH * _SCH
_ZB = 40
_NH = _N // _NC
_TRASH = _NH
_AGGR = 5120
_ABN = _AGGR // _ZB

_f32 = jnp.float32
_i32 = jnp.int32


def _scatter_body(gather_each, edges_hbm, hs_hbm, out_hbm, idxv, rows, zb,
                  aggsh, sems):
    c = lax.axis_index("c")
    s = lax.axis_index("s")
    pltpu.sync_copy(edges_hbm.at[s], idxv.at[pl.ds(0, _SNCH)])
    pltpu.sync_copy(edges_hbm.at[_NS + s], idxv.at[pl.ds(160, _SNCH)])
    pltpu.sync_copy(hs_hbm.at[pl.ds(_N, _ZB)], zb)

    base = c * _NH

    def remap(j, _):
        r = 160 + j
        for q in range(_SCH // 16):
            sl = pl.ds(q * 16, 16)
            v = idxv[r, sl] - base
            ok = (v >= 0) & (v < _NH)
            idxv[r, sl] = jnp.where(ok, v, _TRASH)
        return 0

    lax.fori_loop(0, _SNCH, remap, 0)

    for k in range(_ABN // _NS):
        b = s * (_ABN // _NS) + k
        pltpu.sync_copy(zb, aggsh.at[pl.ds(b * _ZB, _ZB)])
    plsc.subcore_barrier()

    if not gather_each:
        pltpu.async_copy(hs_hbm.at[idxv.at[0]], rows.at[0], sems.at[0])
        pltpu.make_async_copy(
            hs_hbm.at[idxv.at[0]], rows.at[0], sems.at[0]).wait()

        def dstep(j, _):
            pltpu.sync_copy(rows.at[0], aggsh.at[idxv.at[160 + j]], add=True)
            return 0

        lax.fori_loop(0, _SNCH, dstep, 0)
        plsc.subcore_barrier()

        for k in range(_ABN // _NS):
            b = s * (_ABN // _NS) + k
            pltpu.sync_copy(aggsh.at[pl.ds(b * _ZB, _ZB)], zb)
            pltpu.sync_copy(zb, out_hbm.at[c * _ABN + b])
        return

    pltpu.async_copy(hs_hbm.at[idxv.at[0]], rows.at[0], sems.at[0])
    pltpu.async_copy(hs_hbm.at[idxv.at[1]], rows.at[1], sems.at[1])

    def pair(i, _):
        j0 = 2 * i
        j1 = 2 * i + 1
        pltpu.make_async_copy(
            hs_hbm.at[idxv.at[j0]], rows.at[0], sems.at[0]).wait()
        pltpu.sync_copy(rows.at[0], aggsh.at[idxv.at[160 + j0]], add=True)
        pltpu.async_copy(hs_hbm.at[idxv.at[j0 + 2]], rows.at[0], sems.at[0])
        pltpu.make_async_copy(
            hs_hbm.at[idxv.at[j1]], rows.at[1], sems.at[1]).wait()
        pltpu.sync_copy(rows.at[1], aggsh.at[idxv.at[160 + j1]], add=True)
        pltpu.async_copy(hs_hbm.at[idxv.at[j1 + 2]], rows.at[1], sems.at[1])
        return 0

    lax.fori_loop(0, _SNCH // 2 - 1, pair, 0)
    jl0 = _SNCH - 2
    jl1 = _SNCH - 1
    pltpu.make_async_copy(
        hs_hbm.at[idxv.at[jl0]], rows.at[0], sems.at[0]).wait()
    pltpu.sync_copy(rows.at[0], aggsh.at[idxv.at[160 + jl0]], add=True)
    pltpu.make_async_copy(
        hs_hbm.at[idxv.at[jl1]], rows.at[1], sems.at[1]).wait()
    pltpu.sync_copy(rows.at[1], aggsh.at[idxv.at[160 + jl1]], add=True)
    plsc.subcore_barrier()

    for k in range(_ABN // _NS):
        b = s * (_ABN // _NS) + k
        pltpu.sync_copy(aggsh.at[pl.ds(b * _ZB, _ZB)], zb)
        pltpu.sync_copy(zb, out_hbm.at[c * _ABN + b])


@functools.cache
def _sc_kernels():
    mesh = plsc.VectorSubcoreMesh(core_axis_name="c", subcore_axis_name="s")
    def _mk(gather_each):
        return pl.kernel(
            functools.partial(_scatter_body, gather_each),
            out_type=jax.ShapeDtypeStruct((_NC * _ABN, _ZB, _D), _f32),
            mesh=mesh,
            scratch_types=[
                pltpu.VMEM((320, _SCH), _i32),
                pltpu.VMEM((2, _SCH, _D), _f32),
                pltpu.VMEM((_ZB, _D), _f32),
                pltpu.VMEM_SHARED((_AGGR, _D), _f32),
                pltpu.SemaphoreType.DMA((2,)),
            ],
        )

    return _mk(True), _mk(False)


_BR = 1000
_GRID = (_N // _BR,)


def _tc_pre_body(dop, dip, x, hs1, nout, nin):
    do = dop[0, :, 0:1] + 1.0
    di = dip[0, :, 0:1] + 1.0
    no = lax.rsqrt(do)
    ni = lax.rsqrt(di)
    nout[...] = no
    nin[...] = ni
    hs1[...] = x[...] * no


_tc_pre = pl.pallas_call(
    _tc_pre_body,
    grid=_GRID,
    in_specs=[
        pl.BlockSpec((1, _BR, _D), lambda i: (i // 5, i % 5, 0)),
        pl.BlockSpec((1, _BR, _D), lambda i: (i // 5, i % 5, 0)),
        pl.BlockSpec((_BR, _D), lambda i: (i, 0)),
    ],
    out_specs=[
        pl.BlockSpec((_BR, _D), lambda i: (i, 0)),
        pl.BlockSpec((_BR, 1), lambda i: (i, 0)),
        pl.BlockSpec((_BR, 1), lambda i: (i, 0)),
    ],
    out_shape=[
        jax.ShapeDtypeStruct((_N, _D), _f32),
        jax.ShapeDtypeStruct((_N, 1), _f32),
        jax.ShapeDtypeStruct((_N, 1), _f32),
    ],
)


def _tc_mid_body(p, hs, nin, nout, W, b, o):
    agg = (p[0] + hs[...]) * nin[...]
    h = jnp.maximum(jnp.dot(agg, W[...], preferred_element_type=_f32)
                    + b[...], 0.0)
    o[...] = h * nout[...]


_tc_mid = pl.pallas_call(
    _tc_mid_body,
    grid=_GRID,
    in_specs=[
        pl.BlockSpec((1, _BR, _D), lambda i: (i // 5, i % 5, 0)),
        pl.BlockSpec((_BR, _D), lambda i: (i, 0)),
        pl.BlockSpec((_BR, 1), lambda i: (i, 0)),
        pl.BlockSpec((_BR, 1), lambda i: (i, 0)),
        pl.BlockSpec((_D, _D), lambda i: (0, 0)),
        pl.BlockSpec((1, _D), lambda i: (0, 0)),
    ],
    out_specs=pl.BlockSpec((_BR, _D), lambda i: (i, 0)),
    out_shape=jax.ShapeDtypeStruct((_N, _D), _f32),
)


def _tc_final_body(p, hs, nin, W4, b4, out, hscr):
    pc = jnp.concatenate([p[0, :_NH], p[1, :_NH]], axis=0)
    agg = (pc + hs[...]) * nin[...]
    h = jnp.maximum(jnp.dot(agg, W4[...], preferred_element_type=_f32)
                    + b4[...], 0.0)
    hscr[...] = h
    m = jnp.max(h, axis=1, keepdims=True)
    iota = lax.broadcasted_iota(_i32, (_N, 1), 0)
    neg = _f32(-jnp.inf)

    mw = m
    rows = []
    for _ in range(_K):
        mx = jnp.max(mw)
        idx = jnp.min(jnp.where(mw == mx, iota, _N))
        rows.append(hscr[pl.ds(idx, 1), :])
        mw = jnp.where(iota == idx, neg, mw)
    v = jnp.concatenate(rows, axis=0)

    lane = lax.broadcasted_iota(_i32, (_K, _H4), 1)
    even = (lane % 2) == 0
    for pss in range(_H4):
        r = jnp.concatenate([v[:, 1:], v[:, -1:]], axis=1)
        l = jnp.concatenate([v[:, 0:1], v[:, :-1]], axis=1)
        if pss % 2 == 0:
            v = jnp.where(even, jnp.minimum(v, r), jnp.maximum(v, l))
        else:
            start = jnp.logical_not(even) & (lane < _H4 - 1)
            end = even & (lane > 0)
            v = jnp.where(start, jnp.minimum(v, r),
                          jnp.where(end, jnp.maximum(v, l), v))
    out[...] = v


_tc_final = pl.pallas_call(
    _tc_final_body,
    out_shape=jax.ShapeDtypeStruct((_K, _H4), _f32),
    scratch_shapes=[pltpu.VMEM((_N, _H4), _f32)],
)


def kernel(x, edge_index, W1, b1, W2, b2, W3, b3, W4, b4):
    npad = _EPAD - _E
    srcp = jnp.concatenate([edge_index[0], jnp.zeros((npad,), _i32)])
    dstp = jnp.concatenate([edge_index[1], jnp.full((npad,), _N, _i32)])
    edges_s = jnp.concatenate([srcp, dstp]).reshape(2 * _NS, _SNCH, _SCH)
    edges_w = jnp.concatenate([dstp, srcp]).reshape(2 * _NS, _SNCH, _SCH)

    _sc_scatter, _sc_scatter_deg = _sc_kernels()
    zrows = jnp.zeros((_ZB, _D), _f32)

    def _scat(hs, edges):
        return _sc_scatter(
            edges, jnp.concatenate([hs, zrows])).reshape(_NC, _AGGR, _D)

    ones_pad = jnp.concatenate([jnp.ones((_N, _D), _f32), zrows])
    dgi = _sc_scatter_deg(edges_s, ones_pad).reshape(_NC, _AGGR, _D)
    dgo = _sc_scatter_deg(edges_w, ones_pad).reshape(_NC, _AGGR, _D)
    hs1, nout, nin = _tc_pre(dgo, dgi, x)

    p1 = _scat(hs1, edges_s)
    hs2 = _tc_mid(p1, hs1, nin, nout, W1, b1.reshape(1, _D))

    p2 = _scat(hs2, edges_s)
    hs3 = _tc_mid(p2, hs2, nin, nout, W2, b2.reshape(1, _D))

    p3 = _scat(hs3, edges_s)
    hs4 = _tc_mid(p3, hs3, nin, nout, W3, b3.reshape(1, _D))

    p4 = _scat(hs4, edges_s)
    pooled = _tc_final(p4, hs4, nin, W4, b4.reshape(1, _H4))
    return pooled.reshape(1, _K * _H4)
